# baseline (speedup 1.0000x reference)
import functools

import jax
import jax.numpy as jnp
from jax import lax
from jax.experimental import pallas as pl
from jax.experimental.pallas import tpu as pltpu
from jax.experimental.pallas import tpu_sc as plsc

DIM = 64
B = 16384
NC = 2
NS = 16
L = 16
NW = NC * NS
BPW = B // NW
CHUNK = 128
NCH = BPW // CHUNK
NTILES = BPW // L


def _rsqrt(x):
    i = lax.bitcast_convert_type(x, jnp.int32)
    i = jnp.int32(0x5F3759DF) - lax.shift_right_logical(i, 1)
    y = lax.bitcast_convert_type(i, jnp.float32)
    for _ in range(3):
        y = y * (1.5 - 0.5 * x * y * y)
    return y


def _skipgram_body(wt_hbm, wc_hbm, tidx_hbm, cidx_hbm, wsplat_hbm, b_hbm,
                   out_hbm, tidx_v, cidx_v, trows_v, crows_v, wsplat_v,
                   bias_v, out_v, sem):
    wid = lax.axis_index("s") * NC + lax.axis_index("c")

    pltpu.sync_copy(tidx_hbm.at[pl.ds(wid * NCH, NCH)], tidx_v)
    pltpu.sync_copy(cidx_hbm.at[pl.ds(wid * NCH, NCH)], cidx_v)
    pltpu.sync_copy(wsplat_hbm, wsplat_v)
    pltpu.sync_copy(b_hbm, bias_v)

    copies = []
    for j in range(NCH):
        copies.append(pltpu.async_copy(
            wt_hbm.at[tidx_v.at[j]],
            trows_v.at[pl.ds(j * CHUNK, CHUNK)], sem))
        copies.append(pltpu.async_copy(
            wc_hbm.at[cidx_v.at[j]],
            crows_v.at[pl.ds(j * CHUNK, CHUNK)], sem))
    for c in copies:
        c.wait()

    lanes = lax.iota(jnp.int32, L)
    bias = bias_v[...]
    def tile_body(i, carry):
        rows = i * L + lanes
        acc_tt = jnp.zeros((L,), jnp.float32)
        acc_cc = jnp.zeros((L,), jnp.float32)
        acc_tcw = jnp.zeros((L,), jnp.float32)
        for d in range(DIM):
            col = jnp.full((L,), d, jnp.int32)
            t = plsc.load_gather(trows_v, [rows, col])
            c = plsc.load_gather(crows_v, [rows, col])
            w = wsplat_v[d]
            acc_tt = acc_tt + t * t
            acc_cc = acc_cc + c * c
            acc_tcw = acc_tcw + t * c * w
        nt = acc_tt * _rsqrt(acc_tt)
        nc_ = acc_cc * _rsqrt(acc_cc)
        st = jnp.where(nt > 1.0, 1.0 / (nt + 1e-7), 1.0)
        sc = jnp.where(nc_ > 1.0, 1.0 / (nc_ + 1e-7), 1.0)
        logit = acc_tcw * st * sc + bias
        out_v[pl.ds(i * L, L)] = 1.0 / (1.0 + jnp.exp(-logit))
        return carry

    lax.fori_loop(0, NTILES, tile_body, 0)
    pltpu.sync_copy(out_v, out_hbm.at[pl.ds(wid * BPW, BPW)])


@jax.jit
def _skipgram(W_target, W_context, wsplat, bias, tidx, cidx):
    mesh = plsc.VectorSubcoreMesh(core_axis_name="c", subcore_axis_name="s",
                                  num_cores=NC, num_subcores=NS)
    return pl.kernel(
        _skipgram_body,
        out_type=jax.ShapeDtypeStruct((B,), jnp.float32),
        mesh=mesh,
        scratch_types=[
            pltpu.VMEM((NCH, CHUNK), jnp.int32),
            pltpu.VMEM((NCH, CHUNK), jnp.int32),
            pltpu.VMEM((BPW, DIM), jnp.float32),
            pltpu.VMEM((BPW, DIM), jnp.float32),
            pltpu.VMEM((DIM, L), jnp.float32),
            pltpu.VMEM((L,), jnp.float32),
            pltpu.VMEM((BPW,), jnp.float32),
            pltpu.SemaphoreType.DMA,
        ],
        compiler_params=pltpu.CompilerParams(needs_layout_passes=False,
                                             use_tc_tiling_on_sc=False),
        name="skipgram_sc",
    )(W_target, W_context, tidx, cidx, wsplat, bias)


def kernel(W_target, W_context, lin_w, lin_b, target_tensor, context_tensor):
    wsplat = jnp.broadcast_to(lin_w[0][:, None], (DIM, L))
    bias = jnp.broadcast_to(lin_b, (L,))
    tidx = target_tensor.reshape(B // CHUNK, CHUNK)
    cidx = context_tensor.reshape(B // CHUNK, CHUNK)
    return _skipgram(W_target, W_context, wsplat, bias, tidx, cidx)


# speedup vs baseline: 1.5481x; 1.5481x over previous
import jax
import jax.numpy as jnp
from jax import lax
from jax.experimental import pallas as pl
from jax.experimental.pallas import tpu as pltpu
from jax.experimental.pallas import tpu_sc as plsc

DIM = 64
B = 16384
NC = 2
NS = 16
L = 16
NW = NC * NS
BPW = B // NW
NCHUNK = 2
CPW = BPW // NCHUNK
CTILES = CPW // L


def _rsqrt(x):
    i = lax.bitcast_convert_type(x, jnp.int32)
    i = jnp.int32(0x5F3759DF) - lax.shift_right_logical(i, 1)
    y = lax.bitcast_convert_type(i, jnp.float32)
    for _ in range(3):
        y = y * (1.5 - 0.5 * x * y * y)
    return y


def _skipgram_body(wt_hbm, wc_hbm, tidx_hbm, cidx_hbm, wsplat_hbm, b_hbm,
                   drain_hbm, out_hbm, tidx_v, cidx_v, wsplat_v, bias_v,
                   trows_v, crows_v, out_v, sem):
    wid = lax.axis_index("s") * NC + lax.axis_index("c")

    pltpu.sync_copy(tidx_hbm.at[pl.ds(wid * BPW, BPW)], tidx_v)
    pltpu.sync_copy(cidx_hbm.at[pl.ds(wid * BPW, BPW)], cidx_v)
    pltpu.sync_copy(wsplat_hbm, wsplat_v)
    pltpu.sync_copy(b_hbm, bias_v)

    lanes = lax.iota(jnp.int32, L)
    bias = bias_v[...]

    for ch in range(NCHUNK):
        base = ch * CPW

        def fire(g, carry):
            vt = tidx_v[pl.ds(base + g * L, L)]
            vc = cidx_v[pl.ds(base + g * L, L)]
            for j in range(L):
                pltpu.async_copy(wt_hbm.at[vt[j]], trows_v.at[g * L + j],
                                 sem)
                pltpu.async_copy(wc_hbm.at[vc[j]], crows_v.at[g * L + j],
                                 sem)
            return carry

        lax.fori_loop(0, CPW // L, fire, 0)
        pltpu.make_async_copy(drain_hbm, trows_v, sem).wait()
        pltpu.make_async_copy(drain_hbm, crows_v, sem).wait()

        def tile_body(i, carry):
            rows = i * L + lanes
            acc_tt = jnp.zeros((L,), jnp.float32)
            acc_cc = jnp.zeros((L,), jnp.float32)
            acc_tcw = jnp.zeros((L,), jnp.float32)
            for d in range(DIM):
                col = jnp.full((L,), d, jnp.int32)
                t = plsc.load_gather(trows_v, [rows, col])
                c = plsc.load_gather(crows_v, [rows, col])
                w = wsplat_v[d]
                acc_tt = acc_tt + t * t
                acc_cc = acc_cc + c * c
                acc_tcw = acc_tcw + t * c * w
            nt = acc_tt * _rsqrt(acc_tt)
            nc_ = acc_cc * _rsqrt(acc_cc)
            st = jnp.where(nt > 1.0, 1.0 / (nt + 1e-7), 1.0)
            sc = jnp.where(nc_ > 1.0, 1.0 / (nc_ + 1e-7), 1.0)
            logit = acc_tcw * st * sc + bias
            out_v[pl.ds(base + i * L, L)] = 1.0 / (1.0 + jnp.exp(-logit))
            return carry

        lax.fori_loop(0, CTILES, tile_body, 0)

    pltpu.sync_copy(out_v, out_hbm.at[pl.ds(wid * BPW, BPW)])


@jax.jit
def _skipgram(W_target, W_context, wsplat, bias, drain, tidx, cidx):
    mesh = plsc.VectorSubcoreMesh(core_axis_name="c", subcore_axis_name="s",
                                  num_cores=NC, num_subcores=NS)
    return pl.kernel(
        _skipgram_body,
        out_type=jax.ShapeDtypeStruct((B,), jnp.float32),
        mesh=mesh,
        scratch_types=[
            pltpu.VMEM((BPW,), jnp.int32),
            pltpu.VMEM((BPW,), jnp.int32),
            pltpu.VMEM((DIM, L), jnp.float32),
            pltpu.VMEM((L,), jnp.float32),
            pltpu.VMEM((CPW, DIM), jnp.float32),
            pltpu.VMEM((CPW, DIM), jnp.float32),
            pltpu.VMEM((BPW,), jnp.float32),
            pltpu.SemaphoreType.DMA,
        ],
        compiler_params=pltpu.CompilerParams(needs_layout_passes=False),
        name="skipgram_sc",
    )(W_target, W_context, tidx, cidx, wsplat, bias, drain)


def kernel(W_target, W_context, lin_w, lin_b, target_tensor, context_tensor):
    wsplat = jnp.broadcast_to(lin_w[0][:, None], (DIM, L))
    bias = jnp.broadcast_to(lin_b, (L,))
    drain = jnp.zeros((CPW, DIM), jnp.float32)
    return _skipgram(W_target, W_context, wsplat, bias, drain,
                     target_tensor, context_tensor)


# speedup vs baseline: 1.5499x; 1.0012x over previous
import jax
import jax.numpy as jnp
from jax import lax
from jax.experimental import pallas as pl
from jax.experimental.pallas import tpu as pltpu
from jax.experimental.pallas import tpu_sc as plsc

DIM = 64
B = 16384
NC = 2
NS = 16
L = 16
NW = NC * NS
BPW = B // NW
NCHUNK = 2
CPW = BPW // NCHUNK
CTILES = CPW // L


def _rsqrt(x):
    i = lax.bitcast_convert_type(x, jnp.int32)
    i = jnp.int32(0x5F3759DF) - lax.shift_right_logical(i, 1)
    y = lax.bitcast_convert_type(i, jnp.float32)
    for _ in range(3):
        y = y * (1.5 - 0.5 * x * y * y)
    return y


def _skipgram_body(wt_hbm, wc_hbm, tidx_hbm, cidx_hbm, wsplat_hbm, b_hbm,
                   drain_hbm, out_hbm, tidx_v, cidx_v, wsplat_v, bias_v,
                   trows_v, crows_v, out_v, sem):
    wid = lax.axis_index("s") * NC + lax.axis_index("c")

    pltpu.sync_copy(tidx_hbm.at[pl.ds(wid * BPW, BPW)], tidx_v)
    pltpu.sync_copy(cidx_hbm.at[pl.ds(wid * BPW, BPW)], cidx_v)
    pltpu.sync_copy(wsplat_hbm, wsplat_v)
    pltpu.sync_copy(b_hbm, bias_v)

    lanes = lax.iota(jnp.int32, L)
    bias = bias_v[...]

    for ch in range(NCHUNK):
        base = ch * CPW

        def fire(g, carry):
            vt = tidx_v[pl.ds(base + g * L, L)]
            vc = cidx_v[pl.ds(base + g * L, L)]
            for j in range(L):
                pltpu.async_copy(wt_hbm.at[vt[j]], trows_v.at[g * L + j],
                                 sem)
                pltpu.async_copy(wc_hbm.at[vc[j]], crows_v.at[g * L + j],
                                 sem)
            return carry

        lax.fori_loop(0, CPW // L, fire, 0)
        pltpu.make_async_copy(drain_hbm, trows_v, sem).wait()
        pltpu.make_async_copy(drain_hbm, crows_v, sem).wait()

        def tile_body(i, carry):
            rows = i * L + lanes
            acc_tt = jnp.zeros((L,), jnp.float32)
            acc_cc = jnp.zeros((L,), jnp.float32)
            acc_tcw = jnp.zeros((L,), jnp.float32)
            for d in range(DIM):
                col = jnp.full((L,), d, jnp.int32)
                t = plsc.load_gather(trows_v, [rows, col])
                c = plsc.load_gather(crows_v, [rows, col])
                w = wsplat_v[d]
                acc_tt = acc_tt + t * t
                acc_cc = acc_cc + c * c
                acc_tcw = acc_tcw + t * c * w
            nt = acc_tt * _rsqrt(acc_tt)
            nc_ = acc_cc * _rsqrt(acc_cc)
            st = jnp.where(nt > 1.0, 1.0 / (nt + 1e-7), 1.0)
            sc = jnp.where(nc_ > 1.0, 1.0 / (nc_ + 1e-7), 1.0)
            logit = acc_tcw * st * sc + bias
            out_v[pl.ds(base + i * L, L)] = 1.0 / (1.0 + jnp.exp(-logit))
            return carry

        lax.fori_loop(0, CTILES, tile_body, 0)

    pltpu.sync_copy(out_v, out_hbm.at[pl.ds(wid * BPW, BPW)])


@jax.jit
def _skipgram(W_target, W_context, wsplat, bias, drain, tidx, cidx):
    mesh = plsc.VectorSubcoreMesh(core_axis_name="c", subcore_axis_name="s",
                                  num_cores=NC, num_subcores=NS)
    return pl.kernel(
        _skipgram_body,
        out_type=jax.ShapeDtypeStruct((B,), jnp.float32),
        mesh=mesh,
        scratch_types=[
            pltpu.VMEM((BPW,), jnp.int32),
            pltpu.VMEM((BPW,), jnp.int32),
            pltpu.VMEM((DIM, L), jnp.float32),
            pltpu.VMEM((L,), jnp.float32),
            pltpu.VMEM((CPW, DIM), jnp.float32),
            pltpu.VMEM((CPW, DIM), jnp.float32),
            pltpu.VMEM((BPW,), jnp.float32),
            pltpu.SemaphoreType.DMA,
        ],
        compiler_params=pltpu.CompilerParams(needs_layout_passes=False,
                                             skip_device_barrier=True),
        name="skipgram_sc",
    )(W_target, W_context, tidx, cidx, wsplat, bias, drain)


def kernel(W_target, W_context, lin_w, lin_b, target_tensor, context_tensor):
    wsplat = jnp.broadcast_to(lin_w[0][:, None], (DIM, L))
    bias = jnp.broadcast_to(lin_b, (L,))
    drain = jnp.zeros((CPW, DIM), jnp.float32)
    return _skipgram(W_target, W_context, wsplat, bias, drain,
                     target_tensor, context_tensor)
